# baseline (speedup 1.0000x reference)
import jax
import jax.numpy as jnp
from jax.experimental import pallas as pl
from jax.experimental.pallas import tpu as pltpu

_B, _T, _C, _H, _W = 2, 8, 32, 24, 24
_K = 8192
_N = _B * _T * _H * _W
_BM = 256
_GRID = _N // _BM


def _vq_kernel(zf_ref, cb_ref, idx_ref, feat_ref, cnorm_ref):
    zf = zf_ref[...]
    cb = cb_ref[...]

    @pl.when(pl.program_id(0) == 0)
    def _():
        cnorm_ref[...] = jnp.sum(cb * cb, axis=1)[None, :]

    cnorm = cnorm_ref[...]
    rnorm = jnp.sum(zf * zf, axis=1, keepdims=True)
    dots = jax.lax.dot_general(
        zf, cb, (((1,), (1,)), ((), ())),
        preferred_element_type=jnp.float32)
    d = rnorm + cnorm - 2.0 * dots
    min_d = jnp.min(d, axis=1, keepdims=True)
    ids = jax.lax.broadcasted_iota(jnp.int32, d.shape, 1)
    idx = jnp.min(jnp.where(d == min_d, ids, jnp.int32(_K)), axis=1)
    onehot = (ids == idx[:, None]).astype(jnp.float32)
    feats = jax.lax.dot_general(
        onehot, cb, (((1,), (0,)), ((), ())),
        preferred_element_type=jnp.float32)
    idx_ref[0, 0, :] = idx
    feat_ref[...] = feats


def kernel(z, codebook):
    c = z.shape[1]
    zf = jnp.transpose(z, (0, 2, 3, 1)).reshape(_N, c)
    idx, feats = pl.pallas_call(
        _vq_kernel,
        grid=(_GRID,),
        in_specs=[
            pl.BlockSpec((_BM, _C), lambda i: (i, 0)),
            pl.BlockSpec((_K, _C), lambda i: (0, 0)),
        ],
        out_specs=[
            pl.BlockSpec((1, 1, _BM), lambda i: (i, 0, 0)),
            pl.BlockSpec((_BM, _C), lambda i: (i, 0)),
        ],
        out_shape=[
            jax.ShapeDtypeStruct((_GRID, 1, _BM), jnp.int32),
            jax.ShapeDtypeStruct((_N, _C), jnp.float32),
        ],
        scratch_shapes=[pltpu.VMEM((1, _K), jnp.float32)],
    )(zf, codebook)
    L = _H * _W
    indices = idx.reshape(_B, _T, L)
    features = feats.reshape(_B, _T, L, c)
    return indices, features


# speedup vs baseline: 1.0119x; 1.0119x over previous
import jax
import jax.numpy as jnp
from jax.experimental import pallas as pl
from jax.experimental.pallas import tpu as pltpu

_B, _T, _C, _H, _W = 2, 8, 32, 24, 24
_K = 8192
_N = _B * _T * _H * _W
_BM = 256
_GRID = _N // _BM


def _cnorm_kernel(cb_ref, cnorm_ref):
    cb = cb_ref[...]
    cnorm_ref[...] = jnp.sum(cb * cb, axis=1)[None, :]


def _vq_kernel(zf_ref, cb_ref, cnorm_ref, idx_ref, feat_ref):
    zf = zf_ref[...]
    cb = cb_ref[...]
    cnorm = cnorm_ref[...]
    rnorm = jnp.sum(zf * zf, axis=1, keepdims=True)
    zf2 = zf + zf
    dots2 = jax.lax.dot_general(
        zf2, cb, (((1,), (1,)), ((), ())),
        preferred_element_type=jnp.float32)
    d = (rnorm + cnorm) - dots2
    min_d = jnp.min(d, axis=1, keepdims=True)
    ids = jax.lax.broadcasted_iota(jnp.int32, d.shape, 1)
    idx = jnp.min(jnp.where(d == min_d, ids, jnp.int32(_K)), axis=1)
    onehot = (ids == idx[:, None]).astype(jnp.float32)
    feats = jax.lax.dot_general(
        onehot, cb, (((1,), (0,)), ((), ())),
        preferred_element_type=jnp.float32)
    idx_ref[0, 0, :] = idx
    feat_ref[...] = feats


def kernel(z, codebook):
    c = z.shape[1]
    zf = jnp.transpose(z, (0, 2, 3, 1)).reshape(_N, c)
    cnorm = pl.pallas_call(
        _cnorm_kernel,
        out_shape=jax.ShapeDtypeStruct((1, _K), jnp.float32),
    )(codebook)
    idx, feats = pl.pallas_call(
        _vq_kernel,
        grid=(_GRID,),
        in_specs=[
            pl.BlockSpec((_BM, _C), lambda i: (i, 0)),
            pl.BlockSpec((_K, _C), lambda i: (0, 0)),
            pl.BlockSpec((1, _K), lambda i: (0, 0)),
        ],
        out_specs=[
            pl.BlockSpec((1, 1, _BM), lambda i: (i, 0, 0)),
            pl.BlockSpec((_BM, _C), lambda i: (i, 0)),
        ],
        out_shape=[
            jax.ShapeDtypeStruct((_GRID, 1, _BM), jnp.int32),
            jax.ShapeDtypeStruct((_N, _C), jnp.float32),
        ],
    )(zf, codebook, cnorm)
    L = _H * _W
    indices = idx.reshape(_B, _T, L)
    features = feats.reshape(_B, _T, L, c)
    return indices, features


# speedup vs baseline: 1.5575x; 1.5393x over previous
import functools
import jax
import jax.numpy as jnp
from jax import lax
from jax.experimental import pallas as pl
from jax.experimental.pallas import tpu as pltpu
from jax.experimental.pallas import tpu_sc as plsc

_B, _T, _C, _H, _W = 2, 8, 32, 24, 24
_K = 8192
_N = _B * _T * _H * _W
_BM = 256
_GRID = _N // _BM

_NC, _NS = 2, 16
_NW = _NC * _NS
_BPW = _N // _NW
_DP = 128


def _cnorm_kernel(cb_ref, cnorm_ref):
    cb = cb_ref[...]
    cnorm_ref[...] = jnp.sum(cb * cb, axis=1)[None, :]


def _argmin_kernel(zf_ref, cb_ref, cnorm_ref, idx_ref):
    zf = zf_ref[...]
    cb = cb_ref[...]
    cnorm = cnorm_ref[...]
    rnorm = jnp.sum(zf * zf, axis=1, keepdims=True)
    zf2 = zf + zf
    dots2 = jax.lax.dot_general(
        zf2, cb, (((1,), (1,)), ((), ())),
        preferred_element_type=jnp.float32)
    d = (rnorm + cnorm) - dots2
    min_d = jnp.min(d, axis=1, keepdims=True)
    ids = jax.lax.broadcasted_iota(jnp.int32, d.shape, 1)
    idx = jnp.min(jnp.where(d == min_d, ids, jnp.int32(_K)), axis=1)
    idx_ref[0, 0, :] = idx


@functools.partial(
    pl.kernel,
    out_type=jax.ShapeDtypeStruct((_N, _DP), jnp.float32),
    mesh=plsc.VectorSubcoreMesh(core_axis_name="c", subcore_axis_name="s"),
    scratch_types=[
        pltpu.VMEM((_BPW,), jnp.int32),
        pltpu.VMEM((_BPW, _DP), jnp.float32),
        pltpu.SemaphoreType.DMA,
    ],
)
def _gather_kernel(cb_hbm, idx_hbm, out_hbm, idx_v, rows_v, sem):
    wid = lax.axis_index("s") * _NC + lax.axis_index("c")
    base = wid * _BPW
    pltpu.sync_copy(idx_hbm.at[pl.ds(base, _BPW)], idx_v)
    pltpu.async_copy(cb_hbm.at[idx_v], rows_v, sem).wait()
    pltpu.sync_copy(rows_v, out_hbm.at[pl.ds(base, _BPW)])


def kernel(z, codebook):
    c = z.shape[1]
    zf = jnp.transpose(z, (0, 2, 3, 1)).reshape(_N, c)
    cnorm = pl.pallas_call(
        _cnorm_kernel,
        out_shape=jax.ShapeDtypeStruct((1, _K), jnp.float32),
    )(codebook)
    idx3 = pl.pallas_call(
        _argmin_kernel,
        grid=(_GRID,),
        in_specs=[
            pl.BlockSpec((_BM, _C), lambda i: (i, 0)),
            pl.BlockSpec((_K, _C), lambda i: (0, 0)),
            pl.BlockSpec((1, _K), lambda i: (0, 0)),
        ],
        out_specs=pl.BlockSpec((1, 1, _BM), lambda i: (i, 0, 0)),
        out_shape=jax.ShapeDtypeStruct((_GRID, 1, _BM), jnp.int32),
    )(zf, codebook, cnorm)
    idx_flat = idx3.reshape(_N)
    cb_pad = jnp.pad(codebook, ((0, 0), (0, _DP - _C)))
    feats = _gather_kernel(cb_pad, idx_flat)[:, :_C]
    L = _H * _W
    indices = idx_flat.reshape(_B, _T, L)
    features = feats.reshape(_B, _T, L, c)
    return indices, features


# speedup vs baseline: 2.1701x; 1.3933x over previous
import functools
import jax
import jax.numpy as jnp
from jax import lax
from jax.experimental import pallas as pl
from jax.experimental.pallas import tpu as pltpu
from jax.experimental.pallas import tpu_sc as plsc

_B, _T, _C, _H, _W = 2, 8, 32, 24, 24
_K = 8192
_N = _B * _T * _H * _W
_BM = 256
_GRID = _N // _BM

_NC, _NS = 2, 16
_NW = _NC * _NS
_BPW = _N // _NW
_DP = 128


def _cnorm_kernel(cb_ref, cnorm_ref):
    cb = cb_ref[...]
    cnorm_ref[...] = jnp.sum(cb * cb, axis=1)[None, :]


_LB = 128
_NT = _K // _LB
_RH = _BM // 2


def _argmin_kernel(zf_ref, cb_ref, cnorm_ref, idx_ref, dscr_ref):
    zf = zf_ref[...]
    cb = cb_ref[...]
    rnorm = jnp.sum(zf * zf, axis=1, keepdims=True)
    zf2 = zf + zf
    dscr_ref[...] = jax.lax.dot_general(
        zf2, cb, (((1,), (1,)), ((), ())),
        preferred_element_type=jnp.float32)
    idsf = jax.lax.broadcasted_iota(
        jnp.int32, (1, _LB), 1).astype(jnp.float32)
    for r in range(_BM // _RH):
        rn = jax.lax.slice_in_dim(rnorm, r * _RH, (r + 1) * _RH, axis=0)
        acc_v = jnp.full((_RH, _LB), jnp.inf, jnp.float32)
        acc_i = jnp.zeros((_RH, _LB), jnp.float32)
        for t in range(_NT):
            cn = cnorm_ref[:, pl.ds(t * _LB, _LB)]
            dt = dscr_ref[pl.ds(r * _RH, _RH), pl.ds(t * _LB, _LB)]
            v = (rn + cn) - dt
            cmp = v < acc_v
            acc_v = jnp.where(cmp, v, acc_v)
            acc_i = jnp.where(cmp, idsf + jnp.float32(t * _LB), acc_i)
        m = jnp.min(acc_v, axis=1, keepdims=True)
        cand = jnp.where(acc_v == m, acc_i, jnp.float32(_K))
        idx = jnp.min(cand, axis=1).astype(jnp.int32)
        idx_ref[0, 0, pl.ds(r * _RH, _RH)] = idx


@functools.partial(
    pl.kernel,
    out_type=jax.ShapeDtypeStruct((_N, _DP), jnp.float32),
    mesh=plsc.VectorSubcoreMesh(core_axis_name="c", subcore_axis_name="s"),
    scratch_types=[
        pltpu.VMEM((_BPW,), jnp.int32),
        pltpu.VMEM((_BPW, _DP), jnp.float32),
        pltpu.SemaphoreType.DMA,
    ],
)
def _gather_kernel(cb_hbm, idx_hbm, out_hbm, idx_v, rows_v, sem):
    wid = lax.axis_index("s") * _NC + lax.axis_index("c")
    base = wid * _BPW
    pltpu.sync_copy(idx_hbm.at[pl.ds(base, _BPW)], idx_v)
    pltpu.async_copy(cb_hbm.at[idx_v], rows_v, sem).wait()
    pltpu.sync_copy(rows_v, out_hbm.at[pl.ds(base, _BPW)])


def kernel(z, codebook):
    c = z.shape[1]
    zf = jnp.transpose(z, (0, 2, 3, 1)).reshape(_N, c)
    cnorm = pl.pallas_call(
        _cnorm_kernel,
        out_shape=jax.ShapeDtypeStruct((1, _K), jnp.float32),
    )(codebook)
    idx3 = pl.pallas_call(
        _argmin_kernel,
        grid=(_GRID,),
        in_specs=[
            pl.BlockSpec((_BM, _C), lambda i: (i, 0)),
            pl.BlockSpec((_K, _C), lambda i: (0, 0)),
            pl.BlockSpec((1, _K), lambda i: (0, 0)),
        ],
        out_specs=pl.BlockSpec((1, 1, _BM), lambda i: (i, 0, 0)),
        out_shape=jax.ShapeDtypeStruct((_GRID, 1, _BM), jnp.int32),
        scratch_shapes=[pltpu.VMEM((_BM, _K), jnp.float32)],
    )(zf, codebook, cnorm)
    idx_flat = idx3.reshape(_N)
    cb_pad = jnp.pad(codebook, ((0, 0), (0, _DP - _C)))
    feats = _gather_kernel(cb_pad, idx_flat)[:, :_C]
    L = _H * _W
    indices = idx_flat.reshape(_B, _T, L)
    features = feats.reshape(_B, _T, L, c)
    return indices, features


# speedup vs baseline: 2.4085x; 1.1099x over previous
import functools
import jax
import jax.numpy as jnp
from jax import lax
from jax.experimental import pallas as pl
from jax.experimental.pallas import tpu as pltpu
from jax.experimental.pallas import tpu_sc as plsc

_B, _T, _C, _H, _W = 2, 8, 32, 24, 24
_K = 8192
_N = _B * _T * _H * _W
_BM = 512
_GRID = _N // _BM

_NC, _NS = 2, 16
_NW = _NC * _NS
_BPW = _N // _NW
_DP = 128


_LB = 128
_NT = _K // _LB
_RH = 128


def _argmin_kernel(zf_ref, cbt_ref, idx_ref, cnorm_ref, dscr_ref):
    @pl.when(pl.program_id(0) == 0)
    def _():
        cbt0 = cbt_ref[...]
        cnorm_ref[...] = jnp.sum(cbt0 * cbt0, axis=0, keepdims=True)

    zf = zf_ref[...]
    rnorm = jnp.sum(zf * zf, axis=1, keepdims=True)
    zf2 = zf + zf
    dscr_ref[...] = jax.lax.dot_general(
        zf2, cbt_ref[...], (((1,), (0,)), ((), ())),
        preferred_element_type=jnp.float32)
    idsf = jax.lax.broadcasted_iota(
        jnp.int32, (1, _LB), 1).astype(jnp.float32)
    for r in range(_BM // _RH):
        rn = jax.lax.slice_in_dim(rnorm, r * _RH, (r + 1) * _RH, axis=0)
        acc_v = jnp.full((_RH, _LB), jnp.inf, jnp.float32)
        acc_i = jnp.zeros((_RH, _LB), jnp.float32)
        for t in range(_NT):
            cn = cnorm_ref[:, pl.ds(t * _LB, _LB)]
            dt = dscr_ref[pl.ds(r * _RH, _RH), pl.ds(t * _LB, _LB)]
            v = (rn + cn) - dt
            cmp = v < acc_v
            acc_v = jnp.where(cmp, v, acc_v)
            acc_i = jnp.where(cmp, idsf + jnp.float32(t * _LB), acc_i)
        m = jnp.min(acc_v, axis=1, keepdims=True)
        cand = jnp.where(acc_v == m, acc_i, jnp.float32(_K))
        idx = jnp.min(cand, axis=1).astype(jnp.int32)
        idx_ref[0, 0, pl.ds(r * _RH, _RH)] = idx


@functools.partial(
    pl.kernel,
    out_type=jax.ShapeDtypeStruct((_N, _DP), jnp.float32),
    mesh=plsc.VectorSubcoreMesh(core_axis_name="c", subcore_axis_name="s"),
    scratch_types=[
        pltpu.VMEM((_BPW,), jnp.int32),
        pltpu.VMEM((_BPW, _DP), jnp.float32),
        pltpu.SemaphoreType.DMA,
    ],
)
def _gather_kernel(cb_hbm, idx_hbm, out_hbm, idx_v, rows_v, sem):
    wid = lax.axis_index("s") * _NC + lax.axis_index("c")
    base = wid * _BPW
    pltpu.sync_copy(idx_hbm.at[pl.ds(base, _BPW)], idx_v)
    pltpu.async_copy(cb_hbm.at[idx_v], rows_v, sem).wait()
    pltpu.sync_copy(rows_v, out_hbm.at[pl.ds(base, _BPW)])


def kernel(z, codebook):
    c = z.shape[1]
    zf = jnp.transpose(z, (0, 2, 3, 1)).reshape(_N, c)
    cbt = jnp.transpose(codebook, (1, 0))
    idx3 = pl.pallas_call(
        _argmin_kernel,
        grid=(_GRID,),
        in_specs=[
            pl.BlockSpec((_BM, _C), lambda i: (i, 0)),
            pl.BlockSpec((_C, _K), lambda i: (0, 0)),
        ],
        out_specs=pl.BlockSpec((1, 1, _BM), lambda i: (i, 0, 0)),
        out_shape=jax.ShapeDtypeStruct((_GRID, 1, _BM), jnp.int32),
        scratch_shapes=[
            pltpu.VMEM((1, _K), jnp.float32),
            pltpu.VMEM((_BM, _K), jnp.float32),
        ],
    )(zf, cbt)
    idx_flat = idx3.reshape(_N)
    cb_pad = jnp.pad(codebook, ((0, 0), (0, _DP - _C)))
    feats = _gather_kernel(cb_pad, idx_flat)[:, :_C]
    L = _H * _W
    indices = idx_flat.reshape(_B, _T, L)
    features = feats.reshape(_B, _T, L, c)
    return indices, features
